# baseline (speedup 1.0000x reference)
import functools

import jax
import jax.numpy as jnp
from jax.experimental import pallas as pl
from jax.experimental.pallas import tpu as pltpu

N_ROWS = 65536
IN_DIM = 512
H1, H2 = 256, 128
LATENT = 3
SINDY_DIM = 22

P_CORES = 2
BLK_FWD = 512
BLK_STREAM = 1024

_F32 = jnp.float32
_BF16 = jnp.bfloat16


def _dot(a, b):
    return jnp.dot(a, b, preferred_element_type=_F32)


def _sindy_terms(zc):
    d = len(zc)
    ones = jnp.ones_like(zc[0])
    cols = [ones for _ in range(d)]
    cols += [zc[i] for i in range(d)]
    for i in range(d):
        for j in range(i, d):
            cols.append(zc[i] * zc[j])
    for i in range(d):
        for j in range(i, d):
            for k in range(j, d):
                cols.append(zc[i] * zc[j] * zc[k])
    return cols


def _fwd_kernel(x_ref, ew0, eb0, ew1, eb1, ew2, eb2,
                dw0, db0, dw1, db1, dw2, db2, Ew, Eb,
                z_ref, xb_ref, dzb_ref, ge_ref, gd_ref):
    j = pl.program_id(1)

    x = x_ref[...]
    h0 = jax.nn.sigmoid(_dot(x, ew0[...]) + eb0[...])
    g0 = h0 * (1.0 - h0)
    h0b = h0.astype(_BF16)
    h1 = jax.nn.sigmoid(_dot(h0b, ew1[...]) + eb1[...])
    g1 = h1 * (1.0 - h1)
    z = _dot(h1.astype(_BF16), ew2[...]) + eb2[...]
    z_ref[...] = z

    ge_blk = jax.lax.dot_general(g0.astype(_BF16), g1.astype(_BF16),
                                 (((0,), (0,)), ((), ())),
                                 preferred_element_type=_F32)

    @pl.when(j == 0)
    def _():
        ge_ref[...] = ge_blk[None]

    @pl.when(j != 0)
    def _():
        ge_ref[...] += ge_blk[None]

    zc = [z[:, i:i + 1] for i in range(LATENT)]
    terms = _sindy_terms(zc)
    acc = jnp.broadcast_to(Eb[...], z.shape)
    for t, term in enumerate(terms):
        acc = acc + term * Ew[t, :]
    dzb_ref[...] = acc

    hd0 = jax.nn.sigmoid(_dot(z.astype(_BF16), dw0[...]) + db0[...])
    gd0 = hd0 * (1.0 - hd0)
    hd1 = jax.nn.sigmoid(_dot(hd0.astype(_BF16), dw1[...]) + db1[...])
    gd1 = hd1 * (1.0 - hd1)
    xb_ref[...] = _dot(hd1.astype(_BF16), dw2[...]) + db2[...]

    gd_blk = jax.lax.dot_general(gd0.astype(_BF16), gd1.astype(_BF16),
                                 (((0,), (0,)), ((), ())),
                                 preferred_element_type=_F32)

    @pl.when(j == 0)
    def _():
        gd_ref[...] = gd_blk[None]

    @pl.when(j != 0)
    def _():
        gd_ref[...] += gd_blk[None]


def _jac_kernel(ge_ref, gd_ref, ew0, ew1, ew2, dw0, dw1, dw2,
                jet_ref, jdt_ref):
    inv_n = _F32(1.0 / N_ROWS)
    ge = (ge_ref[0] + ge_ref[1]) * inv_n
    jet_ref[...] = _dot(ew0[...], _dot(ew1[...] * ge, ew2[...])).astype(_BF16)
    gd = (gd_ref[0] + gd_ref[1]) * inv_n
    jdt_ref[...] = _dot(_dot(dw0[...], dw1[...] * gd), dw2[...]).astype(_BF16)


def _stream_kernel(dx_ref, dzb_ref, jet_ref, jdt_ref, dz_ref, dxb_ref):
    dz_ref[...] = _dot(dx_ref[...], jet_ref[...])
    dxb_ref[...] = _dot(dzb_ref[...], jdt_ref[...])


def _full(shape):
    return pl.BlockSpec(shape, lambda *_: tuple(0 for _ in shape))


def kernel(x, dx, ddx, enc_w0, enc_b0, enc_w1, enc_b1, enc_w2, enc_b2,
           dec_w0, dec_b0, dec_w1, dec_b1, dec_w2, dec_b2, E_w, E_b,
           interpret=False):
    del ddx

    n = x.shape[0]
    x = x.astype(_BF16)
    dx = dx.astype(_BF16)
    ew0b, ew1b, ew2b = (w.astype(_BF16) for w in (enc_w0, enc_w1, enc_w2))
    dw0b, dw1b, dw2b = (w.astype(_BF16) for w in (dec_w0, dec_w1, dec_w2))
    jf = n // (P_CORES * BLK_FWD)
    row = lambda i, j: (i * jf + j, 0)

    z, xb, dzb, ge_parts, gd_parts = pl.pallas_call(
        _fwd_kernel,
        grid=(P_CORES, jf),
        in_specs=[
            pl.BlockSpec((BLK_FWD, IN_DIM), row),
            _full((IN_DIM, H1)), _full((H1,)),
            _full((H1, H2)), _full((H2,)),
            _full((H2, LATENT)), _full((LATENT,)),
            _full((LATENT, H2)), _full((H2,)),
            _full((H2, H1)), _full((H1,)),
            _full((H1, IN_DIM)), _full((IN_DIM,)),
            _full((SINDY_DIM, LATENT)), _full((LATENT,)),
        ],
        out_specs=[
            pl.BlockSpec((BLK_FWD, LATENT), row),
            pl.BlockSpec((BLK_FWD, IN_DIM), row),
            pl.BlockSpec((BLK_FWD, LATENT), row),
            pl.BlockSpec((1, H1, H2), lambda i, j: (i, 0, 0)),
            pl.BlockSpec((1, H2, H1), lambda i, j: (i, 0, 0)),
        ],
        out_shape=[
            jax.ShapeDtypeStruct((n, LATENT), _F32),
            jax.ShapeDtypeStruct((n, IN_DIM), _F32),
            jax.ShapeDtypeStruct((n, LATENT), _F32),
            jax.ShapeDtypeStruct((P_CORES, H1, H2), _F32),
            jax.ShapeDtypeStruct((P_CORES, H2, H1), _F32),
        ],
        compiler_params=pltpu.CompilerParams(
            dimension_semantics=("parallel", "arbitrary")),
        name="sindy_forward",
        interpret=interpret,
    )(x, ew0b, enc_b0, ew1b, enc_b1, ew2b, enc_b2,
      dw0b, dec_b0, dw1b, dec_b1, dw2b, dec_b2, E_w, E_b)

    jet, jdt = pl.pallas_call(
        _jac_kernel,
        out_shape=[
            jax.ShapeDtypeStruct((IN_DIM, LATENT), _BF16),
            jax.ShapeDtypeStruct((LATENT, IN_DIM), _BF16),
        ],
        name="sindy_mean_jac",
        interpret=interpret,
    )(ge_parts, gd_parts, enc_w0, enc_w1, enc_w2, dec_w0, dec_w1, dec_w2)

    js = n // (P_CORES * BLK_STREAM)
    srow = lambda i, j: (i * js + j, 0)
    dz, dxb = pl.pallas_call(
        _stream_kernel,
        grid=(P_CORES, js),
        in_specs=[
            pl.BlockSpec((BLK_STREAM, IN_DIM), srow),
            pl.BlockSpec((BLK_STREAM, LATENT), srow),
            _full((IN_DIM, LATENT)),
            _full((LATENT, IN_DIM)),
        ],
        out_specs=[
            pl.BlockSpec((BLK_STREAM, LATENT), srow),
            pl.BlockSpec((BLK_STREAM, IN_DIM), srow),
        ],
        out_shape=[
            jax.ShapeDtypeStruct((n, LATENT), _F32),
            jax.ShapeDtypeStruct((n, IN_DIM), _F32),
        ],
        compiler_params=pltpu.CompilerParams(
            dimension_semantics=("parallel", "arbitrary")),
        name="sindy_stream",
        interpret=interpret,
    )(dx, dzb.astype(_BF16), jet, jdt)

    return (z, dz, dzb, xb, dxb)


# speedup vs baseline: 1.2628x; 1.2628x over previous
import functools

import jax
import jax.numpy as jnp
from jax.experimental import pallas as pl
from jax.experimental.pallas import tpu as pltpu

N_ROWS = 65536
IN_DIM = 512
H1, H2 = 256, 128
LATENT = 3
SINDY_DIM = 22

P_CORES = 2
BLK_FWD = 512
BLK_STREAM = 1024

_F32 = jnp.float32
_BF16 = jnp.bfloat16


def _dot(a, b):
    return jnp.dot(a, b, preferred_element_type=_F32)


def _sindy_terms(zc):
    d = len(zc)
    ones = jnp.ones_like(zc[0])
    cols = [ones for _ in range(d)]
    cols += [zc[i] for i in range(d)]
    for i in range(d):
        for j in range(i, d):
            cols.append(zc[i] * zc[j])
    for i in range(d):
        for j in range(i, d):
            for k in range(j, d):
                cols.append(zc[i] * zc[j] * zc[k])
    return cols


def _fwd_kernel(x_ref, ew0, eb0, ew1, eb1, ew2, eb2,
                dw0, db0, dw1, db1, dw2, db2, Ew, Eb,
                z_ref, xb_ref, dzb_ref, ge_ref, gd_ref):
    j = pl.program_id(1)

    x = x_ref[...]
    h0 = jax.nn.sigmoid(_dot(x, ew0[...]) + eb0[...])
    g0 = h0 * (1.0 - h0)
    h1 = jax.nn.sigmoid(_dot(h0, ew1[...]) + eb1[...])
    g1 = h1 * (1.0 - h1)
    z = _dot(h1, ew2[...]) + eb2[...]
    z_ref[...] = z

    ge_blk = jax.lax.dot_general(g0, g1, (((0,), (0,)), ((), ())),
                                 preferred_element_type=_F32)

    @pl.when(j == 0)
    def _():
        ge_ref[...] = ge_blk[None]

    @pl.when(j != 0)
    def _():
        ge_ref[...] += ge_blk[None]

    zc = [z[:, i:i + 1] for i in range(LATENT)]
    terms = _sindy_terms(zc)
    acc = jnp.broadcast_to(Eb[...], z.shape)
    for t, term in enumerate(terms):
        acc = acc + term * Ew[t, :]
    dzb_ref[...] = acc

    hd0 = jax.nn.sigmoid(_dot(z, dw0[...]) + db0[...])
    gd0 = hd0 * (1.0 - hd0)
    hd1 = jax.nn.sigmoid(_dot(hd0, dw1[...]) + db1[...])
    gd1 = hd1 * (1.0 - hd1)
    xb_ref[...] = _dot(hd1, dw2[...]) + db2[...]

    gd_blk = jax.lax.dot_general(gd0, gd1, (((0,), (0,)), ((), ())),
                                 preferred_element_type=_F32)

    @pl.when(j == 0)
    def _():
        gd_ref[...] = gd_blk[None]

    @pl.when(j != 0)
    def _():
        gd_ref[...] += gd_blk[None]


def _jac_kernel(ge_ref, gd_ref, ew0, ew1, ew2, dw0, dw1, dw2,
                jet_ref, jdt_ref):
    inv_n = _F32(1.0 / N_ROWS)
    ge = (ge_ref[0] + ge_ref[1]) * inv_n
    jet_ref[...] = _dot(ew0[...], _dot(ew1[...] * ge, ew2[...]))
    gd = (gd_ref[0] + gd_ref[1]) * inv_n
    jdt_ref[...] = _dot(_dot(dw0[...], dw1[...] * gd), dw2[...])


def _stream_kernel(dx_ref, dzb_ref, jet_ref, jdt_ref, dz_ref, dxb_ref):
    dz_ref[...] = _dot(dx_ref[...], jet_ref[...])
    dxb_ref[...] = _dot(dzb_ref[...], jdt_ref[...])


def _full(shape):
    return pl.BlockSpec(shape, lambda *_: tuple(0 for _ in shape))


def kernel(x, dx, ddx, enc_w0, enc_b0, enc_w1, enc_b1, enc_w2, enc_b2,
           dec_w0, dec_b0, dec_w1, dec_b1, dec_w2, dec_b2, E_w, E_b,
           interpret=False):
    del ddx

    n = x.shape[0]
    jf = n // (P_CORES * BLK_FWD)
    row = lambda i, j: (i * jf + j, 0)

    z, xb, dzb, ge_parts, gd_parts = pl.pallas_call(
        _fwd_kernel,
        grid=(P_CORES, jf),
        in_specs=[
            pl.BlockSpec((BLK_FWD, IN_DIM), row),
            _full((IN_DIM, H1)), _full((H1,)),
            _full((H1, H2)), _full((H2,)),
            _full((H2, LATENT)), _full((LATENT,)),
            _full((LATENT, H2)), _full((H2,)),
            _full((H2, H1)), _full((H1,)),
            _full((H1, IN_DIM)), _full((IN_DIM,)),
            _full((SINDY_DIM, LATENT)), _full((LATENT,)),
        ],
        out_specs=[
            pl.BlockSpec((BLK_FWD, LATENT), row),
            pl.BlockSpec((BLK_FWD, IN_DIM), row),
            pl.BlockSpec((BLK_FWD, LATENT), row),
            pl.BlockSpec((1, H1, H2), lambda i, j: (i, 0, 0)),
            pl.BlockSpec((1, H2, H1), lambda i, j: (i, 0, 0)),
        ],
        out_shape=[
            jax.ShapeDtypeStruct((n, LATENT), _F32),
            jax.ShapeDtypeStruct((n, IN_DIM), _F32),
            jax.ShapeDtypeStruct((n, LATENT), _F32),
            jax.ShapeDtypeStruct((P_CORES, H1, H2), _F32),
            jax.ShapeDtypeStruct((P_CORES, H2, H1), _F32),
        ],
        compiler_params=pltpu.CompilerParams(
            dimension_semantics=("parallel", "arbitrary")),
        name="sindy_forward",
        interpret=interpret,
    )(x, enc_w0, enc_b0, enc_w1, enc_b1, enc_w2, enc_b2,
      dec_w0, dec_b0, dec_w1, dec_b1, dec_w2, dec_b2, E_w, E_b)

    jet, jdt = pl.pallas_call(
        _jac_kernel,
        out_shape=[
            jax.ShapeDtypeStruct((IN_DIM, LATENT), _F32),
            jax.ShapeDtypeStruct((LATENT, IN_DIM), _F32),
        ],
        name="sindy_mean_jac",
        interpret=interpret,
    )(ge_parts, gd_parts, enc_w0, enc_w1, enc_w2, dec_w0, dec_w1, dec_w2)

    js = n // (P_CORES * BLK_STREAM)
    srow = lambda i, j: (i * js + j, 0)
    dz, dxb = pl.pallas_call(
        _stream_kernel,
        grid=(P_CORES, js),
        in_specs=[
            pl.BlockSpec((BLK_STREAM, IN_DIM), srow),
            pl.BlockSpec((BLK_STREAM, LATENT), srow),
            _full((IN_DIM, LATENT)),
            _full((LATENT, IN_DIM)),
        ],
        out_specs=[
            pl.BlockSpec((BLK_STREAM, LATENT), srow),
            pl.BlockSpec((BLK_STREAM, IN_DIM), srow),
        ],
        out_shape=[
            jax.ShapeDtypeStruct((n, LATENT), _F32),
            jax.ShapeDtypeStruct((n, IN_DIM), _F32),
        ],
        compiler_params=pltpu.CompilerParams(
            dimension_semantics=("parallel", "arbitrary")),
        name="sindy_stream",
        interpret=interpret,
    )(dx, dzb, jet, jdt)

    return (z, dz, dzb, xb, dxb)


# speedup vs baseline: 1.6476x; 1.3047x over previous
import functools

import jax
import jax.numpy as jnp
import numpy as np
from jax.experimental import pallas as pl
from jax.experimental.pallas import tpu as pltpu

N_ROWS = 65536
IN_DIM = 512
H1, H2 = 256, 128
LATENT = 3
SINDY_DIM = 22

P_CORES = 2
BLK_FWD = 512
BLK_STREAM = 1024

_F32 = jnp.float32
_BF16 = jnp.bfloat16


def _dot(a, b):
    return jnp.dot(a, b, preferred_element_type=_F32)


def _sindy_selectors():
    factors = [[] for _ in range(LATENT)]
    factors += [[i] for i in range(LATENT)]
    for i in range(LATENT):
        for j in range(i, LATENT):
            factors.append([i, j])
    for i in range(LATENT):
        for j in range(i, LATENT):
            for k in range(j, LATENT):
                factors.append([i, j, k])
    S = np.zeros((3, LATENT, SINDY_DIM), np.float32)
    b = np.zeros((3, SINDY_DIM), np.float32)
    for t, f in enumerate(factors):
        for m in range(3):
            if m < len(f):
                S[m, f[m], t] = 1.0
            else:
                b[m, t] = 1.0
    return S, b


def _fwd_kernel(x_ref, ew0, eb0, ew1, eb1, ew2, eb2,
                dw0, db0, dw1, db1, dw2, db2, Ew, Eb, S_ref, sb_ref,
                z_ref, xb_ref, dzb_ref, ge_ref, gd_ref):
    j = pl.program_id(1)

    x = x_ref[...]
    h0 = jax.nn.sigmoid(_dot(x, ew0[...]) + eb0[...])
    g0 = h0 * (1.0 - h0)
    h1 = jax.nn.sigmoid(_dot(h0, ew1[...]) + eb1[...])
    g1 = h1 * (1.0 - h1)
    z = _dot(h1, ew2[...]) + eb2[...]
    z_ref[...] = z

    ge_blk = jax.lax.dot_general(g0, g1, (((0,), (0,)), ((), ())),
                                 preferred_element_type=_F32)

    @pl.when(j == 0)
    def _():
        ge_ref[...] = ge_blk[None]

    @pl.when(j != 0)
    def _():
        ge_ref[...] += ge_blk[None]

    p0 = _dot(z, S_ref[0]) + sb_ref[0, :]
    p1 = _dot(z, S_ref[1]) + sb_ref[1, :]
    p2 = _dot(z, S_ref[2]) + sb_ref[2, :]
    theta = p0 * p1 * p2
    dzb_ref[...] = _dot(theta, Ew[...]) + Eb[...]

    hd0 = jax.nn.sigmoid(_dot(z, dw0[...]) + db0[...])
    gd0 = hd0 * (1.0 - hd0)
    hd1 = jax.nn.sigmoid(_dot(hd0, dw1[...]) + db1[...])
    gd1 = hd1 * (1.0 - hd1)
    xb_ref[...] = _dot(hd1, dw2[...]) + db2[...]

    gd_blk = jax.lax.dot_general(gd0, gd1, (((0,), (0,)), ((), ())),
                                 preferred_element_type=_F32)

    @pl.when(j == 0)
    def _():
        gd_ref[...] = gd_blk[None]

    @pl.when(j != 0)
    def _():
        gd_ref[...] += gd_blk[None]


def _jac_kernel(ge_ref, gd_ref, ew0, ew1, ew2, dw0, dw1, dw2,
                jet_ref, jdt_ref):
    inv_n = _F32(1.0 / N_ROWS)
    ge = (ge_ref[0] + ge_ref[1]) * inv_n
    jet_ref[...] = _dot(ew0[...], _dot(ew1[...] * ge, ew2[...]))
    gd = (gd_ref[0] + gd_ref[1]) * inv_n
    jdt_ref[...] = _dot(_dot(dw0[...], dw1[...] * gd), dw2[...])


def _stream_kernel(dx_ref, dzb_ref, jet_ref, jdt_ref, dz_ref, dxb_ref):
    dz_ref[...] = _dot(dx_ref[...], jet_ref[...])
    dxb_ref[...] = _dot(dzb_ref[...], jdt_ref[...])


def _full(shape):
    return pl.BlockSpec(shape, lambda *_: tuple(0 for _ in shape))


_SINDY_S, _SINDY_B = _sindy_selectors()


def kernel(x, dx, ddx, enc_w0, enc_b0, enc_w1, enc_b1, enc_w2, enc_b2,
           dec_w0, dec_b0, dec_w1, dec_b1, dec_w2, dec_b2, E_w, E_b,
           interpret=False):
    del ddx

    n = x.shape[0]
    jf = n // (P_CORES * BLK_FWD)
    row = lambda i, j: (i * jf + j, 0)

    z, xb, dzb, ge_parts, gd_parts = pl.pallas_call(
        _fwd_kernel,
        grid=(P_CORES, jf),
        in_specs=[
            pl.BlockSpec((BLK_FWD, IN_DIM), row),
            _full((IN_DIM, H1)), _full((H1,)),
            _full((H1, H2)), _full((H2,)),
            _full((H2, LATENT)), _full((LATENT,)),
            _full((LATENT, H2)), _full((H2,)),
            _full((H2, H1)), _full((H1,)),
            _full((H1, IN_DIM)), _full((IN_DIM,)),
            _full((SINDY_DIM, LATENT)), _full((LATENT,)),
            _full((3, LATENT, SINDY_DIM)), _full((3, SINDY_DIM)),
        ],
        out_specs=[
            pl.BlockSpec((BLK_FWD, LATENT), row),
            pl.BlockSpec((BLK_FWD, IN_DIM), row),
            pl.BlockSpec((BLK_FWD, LATENT), row),
            pl.BlockSpec((1, H1, H2), lambda i, j: (i, 0, 0)),
            pl.BlockSpec((1, H2, H1), lambda i, j: (i, 0, 0)),
        ],
        out_shape=[
            jax.ShapeDtypeStruct((n, LATENT), _F32),
            jax.ShapeDtypeStruct((n, IN_DIM), _F32),
            jax.ShapeDtypeStruct((n, LATENT), _F32),
            jax.ShapeDtypeStruct((P_CORES, H1, H2), _F32),
            jax.ShapeDtypeStruct((P_CORES, H2, H1), _F32),
        ],
        compiler_params=pltpu.CompilerParams(
            dimension_semantics=("parallel", "arbitrary")),
        name="sindy_forward",
        interpret=interpret,
    )(x, enc_w0, enc_b0, enc_w1, enc_b1, enc_w2, enc_b2,
      dec_w0, dec_b0, dec_w1, dec_b1, dec_w2, dec_b2, E_w, E_b,
      jnp.asarray(_SINDY_S), jnp.asarray(_SINDY_B))

    jet, jdt = pl.pallas_call(
        _jac_kernel,
        out_shape=[
            jax.ShapeDtypeStruct((IN_DIM, LATENT), _F32),
            jax.ShapeDtypeStruct((LATENT, IN_DIM), _F32),
        ],
        name="sindy_mean_jac",
        interpret=interpret,
    )(ge_parts, gd_parts, enc_w0, enc_w1, enc_w2, dec_w0, dec_w1, dec_w2)

    js = n // (P_CORES * BLK_STREAM)
    srow = lambda i, j: (i * js + j, 0)
    dz, dxb = pl.pallas_call(
        _stream_kernel,
        grid=(P_CORES, js),
        in_specs=[
            pl.BlockSpec((BLK_STREAM, IN_DIM), srow),
            pl.BlockSpec((BLK_STREAM, LATENT), srow),
            _full((IN_DIM, LATENT)),
            _full((LATENT, IN_DIM)),
        ],
        out_specs=[
            pl.BlockSpec((BLK_STREAM, LATENT), srow),
            pl.BlockSpec((BLK_STREAM, IN_DIM), srow),
        ],
        out_shape=[
            jax.ShapeDtypeStruct((n, LATENT), _F32),
            jax.ShapeDtypeStruct((n, IN_DIM), _F32),
        ],
        compiler_params=pltpu.CompilerParams(
            dimension_semantics=("parallel", "arbitrary")),
        name="sindy_stream",
        interpret=interpret,
    )(dx, dzb, jet, jdt)

    return (z, dz, dzb, xb, dxb)


# speedup vs baseline: 1.9810x; 1.2024x over previous
import functools

import jax
import jax.numpy as jnp
import numpy as np
from jax.experimental import pallas as pl
from jax.experimental.pallas import tpu as pltpu

N_ROWS = 65536
IN_DIM = 512
H1, H2 = 256, 128
LATENT = 3
SINDY_DIM = 22

P_CORES = 2
BLK_FWD = 1024
BLK_STREAM = 2048

_F32 = jnp.float32
_BF16 = jnp.bfloat16


def _dot(a, b):
    return jnp.dot(a, b, preferred_element_type=_F32)


def _sindy_selectors():
    factors = [[] for _ in range(LATENT)]
    factors += [[i] for i in range(LATENT)]
    for i in range(LATENT):
        for j in range(i, LATENT):
            factors.append([i, j])
    for i in range(LATENT):
        for j in range(i, LATENT):
            for k in range(j, LATENT):
                factors.append([i, j, k])
    S = np.zeros((3, LATENT, SINDY_DIM), np.float32)
    b = np.zeros((3, SINDY_DIM), np.float32)
    for t, f in enumerate(factors):
        for m in range(3):
            if m < len(f):
                S[m, f[m], t] = 1.0
            else:
                b[m, t] = 1.0
    return S, b


def _fwd_kernel(x_ref, ew0, eb0, ew1, eb1, ew2, eb2,
                dw0, db0, dw1, db1, dw2, db2, Ew, Eb, S_ref, sb_ref,
                z_ref, xb_ref, dzb_ref, ge_ref, gd_ref):
    j = pl.program_id(1)

    x = x_ref[...]
    h0 = jax.nn.sigmoid(_dot(x, ew0[...]) + eb0[...])
    g0 = h0 * (1.0 - h0)
    h1 = jax.nn.sigmoid(_dot(h0, ew1[...]) + eb1[...])
    g1 = h1 * (1.0 - h1)
    z = _dot(h1, ew2[...]) + eb2[...]
    z_ref[...] = z

    ge_blk = jax.lax.dot_general(g0, g1, (((0,), (0,)), ((), ())),
                                 preferred_element_type=_F32)

    @pl.when(j == 0)
    def _():
        ge_ref[...] = ge_blk[None]

    @pl.when(j != 0)
    def _():
        ge_ref[...] += ge_blk[None]

    p0 = _dot(z, S_ref[0]) + sb_ref[0, :]
    p1 = _dot(z, S_ref[1]) + sb_ref[1, :]
    p2 = _dot(z, S_ref[2]) + sb_ref[2, :]
    theta = p0 * p1 * p2
    dzb_ref[...] = _dot(theta, Ew[...]) + Eb[...]

    hd0 = jax.nn.sigmoid(_dot(z, dw0[...]) + db0[...])
    gd0 = hd0 * (1.0 - hd0)
    hd1 = jax.nn.sigmoid(_dot(hd0, dw1[...]) + db1[...])
    gd1 = hd1 * (1.0 - hd1)
    xb_ref[...] = _dot(hd1, dw2[...]) + db2[...]

    gd_blk = jax.lax.dot_general(gd0, gd1, (((0,), (0,)), ((), ())),
                                 preferred_element_type=_F32)

    @pl.when(j == 0)
    def _():
        gd_ref[...] = gd_blk[None]

    @pl.when(j != 0)
    def _():
        gd_ref[...] += gd_blk[None]


def _jac_kernel(ge_ref, gd_ref, ew0, ew1, ew2, dw0, dw1, dw2,
                jet_ref, jdt_ref):
    inv_n = _F32(1.0 / N_ROWS)
    ge = (ge_ref[0] + ge_ref[1]) * inv_n
    jet_ref[...] = _dot(ew0[...], _dot(ew1[...] * ge, ew2[...]))
    gd = (gd_ref[0] + gd_ref[1]) * inv_n
    jdt_ref[...] = _dot(_dot(dw0[...], dw1[...] * gd), dw2[...])


def _stream_kernel(dx_ref, dzb_ref, jet_ref, jdt_ref, dz_ref, dxb_ref):
    dz_ref[...] = _dot(dx_ref[...], jet_ref[...])
    dxb_ref[...] = _dot(dzb_ref[...], jdt_ref[...])


def _full(shape):
    return pl.BlockSpec(shape, lambda *_: tuple(0 for _ in shape))


_SINDY_S, _SINDY_B = _sindy_selectors()


def kernel(x, dx, ddx, enc_w0, enc_b0, enc_w1, enc_b1, enc_w2, enc_b2,
           dec_w0, dec_b0, dec_w1, dec_b1, dec_w2, dec_b2, E_w, E_b,
           interpret=False):
    del ddx

    n = x.shape[0]
    jf = n // (P_CORES * BLK_FWD)
    row = lambda i, j: (i * jf + j, 0)

    z, xb, dzb, ge_parts, gd_parts = pl.pallas_call(
        _fwd_kernel,
        grid=(P_CORES, jf),
        in_specs=[
            pl.BlockSpec((BLK_FWD, IN_DIM), row),
            _full((IN_DIM, H1)), _full((H1,)),
            _full((H1, H2)), _full((H2,)),
            _full((H2, LATENT)), _full((LATENT,)),
            _full((LATENT, H2)), _full((H2,)),
            _full((H2, H1)), _full((H1,)),
            _full((H1, IN_DIM)), _full((IN_DIM,)),
            _full((SINDY_DIM, LATENT)), _full((LATENT,)),
            _full((3, LATENT, SINDY_DIM)), _full((3, SINDY_DIM)),
        ],
        out_specs=[
            pl.BlockSpec((BLK_FWD, LATENT), row),
            pl.BlockSpec((BLK_FWD, IN_DIM), row),
            pl.BlockSpec((BLK_FWD, LATENT), row),
            pl.BlockSpec((1, H1, H2), lambda i, j: (i, 0, 0)),
            pl.BlockSpec((1, H2, H1), lambda i, j: (i, 0, 0)),
        ],
        out_shape=[
            jax.ShapeDtypeStruct((n, LATENT), _F32),
            jax.ShapeDtypeStruct((n, IN_DIM), _F32),
            jax.ShapeDtypeStruct((n, LATENT), _F32),
            jax.ShapeDtypeStruct((P_CORES, H1, H2), _F32),
            jax.ShapeDtypeStruct((P_CORES, H2, H1), _F32),
        ],
        compiler_params=pltpu.CompilerParams(
            dimension_semantics=("parallel", "arbitrary")),
        name="sindy_forward",
        interpret=interpret,
    )(x, enc_w0, enc_b0, enc_w1, enc_b1, enc_w2, enc_b2,
      dec_w0, dec_b0, dec_w1, dec_b1, dec_w2, dec_b2, E_w, E_b,
      jnp.asarray(_SINDY_S), jnp.asarray(_SINDY_B))

    jet, jdt = pl.pallas_call(
        _jac_kernel,
        out_shape=[
            jax.ShapeDtypeStruct((IN_DIM, LATENT), _F32),
            jax.ShapeDtypeStruct((LATENT, IN_DIM), _F32),
        ],
        name="sindy_mean_jac",
        interpret=interpret,
    )(ge_parts, gd_parts, enc_w0, enc_w1, enc_w2, dec_w0, dec_w1, dec_w2)

    js = n // (P_CORES * BLK_STREAM)
    srow = lambda i, j: (i * js + j, 0)
    dz, dxb = pl.pallas_call(
        _stream_kernel,
        grid=(P_CORES, js),
        in_specs=[
            pl.BlockSpec((BLK_STREAM, IN_DIM), srow),
            pl.BlockSpec((BLK_STREAM, LATENT), srow),
            _full((IN_DIM, LATENT)),
            _full((LATENT, IN_DIM)),
        ],
        out_specs=[
            pl.BlockSpec((BLK_STREAM, LATENT), srow),
            pl.BlockSpec((BLK_STREAM, IN_DIM), srow),
        ],
        out_shape=[
            jax.ShapeDtypeStruct((n, LATENT), _F32),
            jax.ShapeDtypeStruct((n, IN_DIM), _F32),
        ],
        compiler_params=pltpu.CompilerParams(
            dimension_semantics=("parallel", "arbitrary")),
        name="sindy_stream",
        interpret=interpret,
    )(dx, dzb, jet, jdt)

    return (z, dz, dzb, xb, dxb)


# speedup vs baseline: 2.1156x; 1.0679x over previous
import functools

import jax
import jax.numpy as jnp
import numpy as np
from jax.experimental import pallas as pl
from jax.experimental.pallas import tpu as pltpu

N_ROWS = 65536
IN_DIM = 512
H1, H2 = 256, 128
LATENT = 3
SINDY_DIM = 22

P_CORES = 2
BLK_FWD = 2048
BLK_STREAM = 4096

_F32 = jnp.float32
_BF16 = jnp.bfloat16


def _dot(a, b):
    return jnp.dot(a, b, preferred_element_type=_F32)


def _sindy_selectors():
    factors = [[] for _ in range(LATENT)]
    factors += [[i] for i in range(LATENT)]
    for i in range(LATENT):
        for j in range(i, LATENT):
            factors.append([i, j])
    for i in range(LATENT):
        for j in range(i, LATENT):
            for k in range(j, LATENT):
                factors.append([i, j, k])
    S = np.zeros((3, LATENT, SINDY_DIM), np.float32)
    b = np.zeros((3, SINDY_DIM), np.float32)
    for t, f in enumerate(factors):
        for m in range(3):
            if m < len(f):
                S[m, f[m], t] = 1.0
            else:
                b[m, t] = 1.0
    return S, b


def _fwd_kernel(x_ref, ew0, eb0, ew1, eb1, ew2, eb2,
                dw0, db0, dw1, db1, dw2, db2, Ew, Eb, S_ref, sb_ref,
                z_ref, xb_ref, dzb_ref, ge_ref, gd_ref):
    j = pl.program_id(1)

    x = x_ref[...]
    h0 = jax.nn.sigmoid(_dot(x, ew0[...]) + eb0[...])
    g0 = h0 * (1.0 - h0)
    h1 = jax.nn.sigmoid(_dot(h0, ew1[...]) + eb1[...])
    g1 = h1 * (1.0 - h1)
    z = _dot(h1, ew2[...]) + eb2[...]
    z_ref[...] = z

    ge_blk = jax.lax.dot_general(g0, g1, (((0,), (0,)), ((), ())),
                                 preferred_element_type=_F32)

    @pl.when(j == 0)
    def _():
        ge_ref[...] = ge_blk[None]

    @pl.when(j != 0)
    def _():
        ge_ref[...] += ge_blk[None]

    p0 = _dot(z, S_ref[0]) + sb_ref[0, :]
    p1 = _dot(z, S_ref[1]) + sb_ref[1, :]
    p2 = _dot(z, S_ref[2]) + sb_ref[2, :]
    theta = p0 * p1 * p2
    dzb_ref[...] = _dot(theta, Ew[...]) + Eb[...]

    hd0 = jax.nn.sigmoid(_dot(z, dw0[...]) + db0[...])
    gd0 = hd0 * (1.0 - hd0)
    hd1 = jax.nn.sigmoid(_dot(hd0, dw1[...]) + db1[...])
    gd1 = hd1 * (1.0 - hd1)
    xb_ref[...] = _dot(hd1, dw2[...]) + db2[...]

    gd_blk = jax.lax.dot_general(gd0, gd1, (((0,), (0,)), ((), ())),
                                 preferred_element_type=_F32)

    @pl.when(j == 0)
    def _():
        gd_ref[...] = gd_blk[None]

    @pl.when(j != 0)
    def _():
        gd_ref[...] += gd_blk[None]


def _jac_kernel(ge_ref, gd_ref, ew0, ew1, ew2, dw0, dw1, dw2,
                jet_ref, jdt_ref):
    inv_n = _F32(1.0 / N_ROWS)
    ge = (ge_ref[0] + ge_ref[1]) * inv_n
    jet_ref[...] = _dot(ew0[...], _dot(ew1[...] * ge, ew2[...]))
    gd = (gd_ref[0] + gd_ref[1]) * inv_n
    jdt_ref[...] = _dot(_dot(dw0[...], dw1[...] * gd), dw2[...])


def _stream_kernel(dx_ref, dzb_ref, jet_ref, jdt_ref, dz_ref, dxb_ref):
    dz_ref[...] = _dot(dx_ref[...], jet_ref[...])
    dxb_ref[...] = _dot(dzb_ref[...], jdt_ref[...])


def _full(shape):
    return pl.BlockSpec(shape, lambda *_: tuple(0 for _ in shape))


_SINDY_S, _SINDY_B = _sindy_selectors()


def kernel(x, dx, ddx, enc_w0, enc_b0, enc_w1, enc_b1, enc_w2, enc_b2,
           dec_w0, dec_b0, dec_w1, dec_b1, dec_w2, dec_b2, E_w, E_b,
           interpret=False):
    del ddx

    n = x.shape[0]
    jf = n // (P_CORES * BLK_FWD)
    row = lambda i, j: (i * jf + j, 0)

    z, xb, dzb, ge_parts, gd_parts = pl.pallas_call(
        _fwd_kernel,
        grid=(P_CORES, jf),
        in_specs=[
            pl.BlockSpec((BLK_FWD, IN_DIM), row),
            _full((IN_DIM, H1)), _full((H1,)),
            _full((H1, H2)), _full((H2,)),
            _full((H2, LATENT)), _full((LATENT,)),
            _full((LATENT, H2)), _full((H2,)),
            _full((H2, H1)), _full((H1,)),
            _full((H1, IN_DIM)), _full((IN_DIM,)),
            _full((SINDY_DIM, LATENT)), _full((LATENT,)),
            _full((3, LATENT, SINDY_DIM)), _full((3, SINDY_DIM)),
        ],
        out_specs=[
            pl.BlockSpec((BLK_FWD, LATENT), row),
            pl.BlockSpec((BLK_FWD, IN_DIM), row),
            pl.BlockSpec((BLK_FWD, LATENT), row),
            pl.BlockSpec((1, H1, H2), lambda i, j: (i, 0, 0)),
            pl.BlockSpec((1, H2, H1), lambda i, j: (i, 0, 0)),
        ],
        out_shape=[
            jax.ShapeDtypeStruct((n, LATENT), _F32),
            jax.ShapeDtypeStruct((n, IN_DIM), _F32),
            jax.ShapeDtypeStruct((n, LATENT), _F32),
            jax.ShapeDtypeStruct((P_CORES, H1, H2), _F32),
            jax.ShapeDtypeStruct((P_CORES, H2, H1), _F32),
        ],
        compiler_params=pltpu.CompilerParams(
            dimension_semantics=("parallel", "arbitrary"),
            vmem_limit_bytes=56 * 1024 * 1024),
        name="sindy_forward",
        interpret=interpret,
    )(x, enc_w0, enc_b0, enc_w1, enc_b1, enc_w2, enc_b2,
      dec_w0, dec_b0, dec_w1, dec_b1, dec_w2, dec_b2, E_w, E_b,
      jnp.asarray(_SINDY_S), jnp.asarray(_SINDY_B))

    jet, jdt = pl.pallas_call(
        _jac_kernel,
        out_shape=[
            jax.ShapeDtypeStruct((IN_DIM, LATENT), _F32),
            jax.ShapeDtypeStruct((LATENT, IN_DIM), _F32),
        ],
        name="sindy_mean_jac",
        interpret=interpret,
    )(ge_parts, gd_parts, enc_w0, enc_w1, enc_w2, dec_w0, dec_w1, dec_w2)

    js = n // (P_CORES * BLK_STREAM)
    srow = lambda i, j: (i * js + j, 0)
    dz, dxb = pl.pallas_call(
        _stream_kernel,
        grid=(P_CORES, js),
        in_specs=[
            pl.BlockSpec((BLK_STREAM, IN_DIM), srow),
            pl.BlockSpec((BLK_STREAM, LATENT), srow),
            _full((IN_DIM, LATENT)),
            _full((LATENT, IN_DIM)),
        ],
        out_specs=[
            pl.BlockSpec((BLK_STREAM, LATENT), srow),
            pl.BlockSpec((BLK_STREAM, IN_DIM), srow),
        ],
        out_shape=[
            jax.ShapeDtypeStruct((n, LATENT), _F32),
            jax.ShapeDtypeStruct((n, IN_DIM), _F32),
        ],
        compiler_params=pltpu.CompilerParams(
            dimension_semantics=("parallel", "arbitrary"),
            vmem_limit_bytes=56 * 1024 * 1024),
        name="sindy_stream",
        interpret=interpret,
    )(dx, dzb, jet, jdt)

    return (z, dz, dzb, xb, dxb)


# speedup vs baseline: 2.1532x; 1.0178x over previous
import functools

import jax
import jax.numpy as jnp
import numpy as np
from jax.experimental import pallas as pl
from jax.experimental.pallas import tpu as pltpu

N_ROWS = 65536
IN_DIM = 512
H1, H2 = 256, 128
LATENT = 3
SINDY_DIM = 22

P_CORES = 2
BLK_FWD = 4096
BLK_STREAM = 4096

_F32 = jnp.float32
_BF16 = jnp.bfloat16


def _dot(a, b):
    return jnp.dot(a, b, preferred_element_type=_F32)


def _sindy_selectors():
    factors = [[] for _ in range(LATENT)]
    factors += [[i] for i in range(LATENT)]
    for i in range(LATENT):
        for j in range(i, LATENT):
            factors.append([i, j])
    for i in range(LATENT):
        for j in range(i, LATENT):
            for k in range(j, LATENT):
                factors.append([i, j, k])
    S = np.zeros((3, LATENT, SINDY_DIM), np.float32)
    b = np.zeros((3, SINDY_DIM), np.float32)
    for t, f in enumerate(factors):
        for m in range(3):
            if m < len(f):
                S[m, f[m], t] = 1.0
            else:
                b[m, t] = 1.0
    return S, b


def _fwd_kernel(x_ref, ew0, eb0, ew1, eb1, ew2, eb2,
                dw0, db0, dw1, db1, dw2, db2, Ew, Eb, S_ref, sb_ref,
                z_ref, xb_ref, dzb_ref, ge_ref, gd_ref):
    j = pl.program_id(1)

    x = x_ref[...]
    h0 = jax.nn.sigmoid(_dot(x, ew0[...]) + eb0[...])
    g0 = h0 * (1.0 - h0)
    h1 = jax.nn.sigmoid(_dot(h0, ew1[...]) + eb1[...])
    g1 = h1 * (1.0 - h1)
    z = _dot(h1, ew2[...]) + eb2[...]
    z_ref[...] = z

    ge_blk = jax.lax.dot_general(g0, g1, (((0,), (0,)), ((), ())),
                                 preferred_element_type=_F32)

    @pl.when(j == 0)
    def _():
        ge_ref[...] = ge_blk[None]

    @pl.when(j != 0)
    def _():
        ge_ref[...] += ge_blk[None]

    p0 = _dot(z, S_ref[0]) + sb_ref[0, :]
    p1 = _dot(z, S_ref[1]) + sb_ref[1, :]
    p2 = _dot(z, S_ref[2]) + sb_ref[2, :]
    theta = p0 * p1 * p2
    dzb_ref[...] = _dot(theta, Ew[...]) + Eb[...]

    hd0 = jax.nn.sigmoid(_dot(z, dw0[...]) + db0[...])
    gd0 = hd0 * (1.0 - hd0)
    hd1 = jax.nn.sigmoid(_dot(hd0, dw1[...]) + db1[...])
    gd1 = hd1 * (1.0 - hd1)
    xb_ref[...] = _dot(hd1, dw2[...]) + db2[...]

    gd_blk = jax.lax.dot_general(gd0, gd1, (((0,), (0,)), ((), ())),
                                 preferred_element_type=_F32)

    @pl.when(j == 0)
    def _():
        gd_ref[...] = gd_blk[None]

    @pl.when(j != 0)
    def _():
        gd_ref[...] += gd_blk[None]


def _jac_kernel(ge_ref, gd_ref, ew0, ew1, ew2, dw0, dw1, dw2,
                jet_ref, jdt_ref):
    inv_n = _F32(1.0 / N_ROWS)
    ge = jnp.sum(ge_ref[...], axis=0) * inv_n
    jet_ref[...] = _dot(ew0[...], _dot(ew1[...] * ge, ew2[...]))
    gd = jnp.sum(gd_ref[...], axis=0) * inv_n
    jdt_ref[...] = _dot(_dot(dw0[...], dw1[...] * gd), dw2[...])


def _stream_kernel(dx_ref, dzb_ref, jet_ref, jdt_ref, dz_ref, dxb_ref):
    dz_ref[...] = _dot(dx_ref[...], jet_ref[...])
    dxb_ref[...] = _dot(dzb_ref[...], jdt_ref[...])


def _full(shape):
    return pl.BlockSpec(shape, lambda *_: tuple(0 for _ in shape))


_SINDY_S, _SINDY_B = _sindy_selectors()


def kernel(x, dx, ddx, enc_w0, enc_b0, enc_w1, enc_b1, enc_w2, enc_b2,
           dec_w0, dec_b0, dec_w1, dec_b1, dec_w2, dec_b2, E_w, E_b,
           interpret=False):
    del ddx

    n = x.shape[0]
    jf = n // (P_CORES * BLK_FWD)
    row = lambda i, j: (i * jf + j, 0)

    z, xb, dzb, ge_parts, gd_parts = pl.pallas_call(
        _fwd_kernel,
        grid=(P_CORES, jf),
        in_specs=[
            pl.BlockSpec((BLK_FWD, IN_DIM), row),
            _full((IN_DIM, H1)), _full((H1,)),
            _full((H1, H2)), _full((H2,)),
            _full((H2, LATENT)), _full((LATENT,)),
            _full((LATENT, H2)), _full((H2,)),
            _full((H2, H1)), _full((H1,)),
            _full((H1, IN_DIM)), _full((IN_DIM,)),
            _full((SINDY_DIM, LATENT)), _full((LATENT,)),
            _full((3, LATENT, SINDY_DIM)), _full((3, SINDY_DIM)),
        ],
        out_specs=[
            pl.BlockSpec((BLK_FWD, LATENT), row),
            pl.BlockSpec((BLK_FWD, IN_DIM), row),
            pl.BlockSpec((BLK_FWD, LATENT), row),
            pl.BlockSpec((1, H1, H2), lambda i, j: (i, 0, 0)),
            pl.BlockSpec((1, H2, H1), lambda i, j: (i, 0, 0)),
        ],
        out_shape=[
            jax.ShapeDtypeStruct((n, LATENT), _F32),
            jax.ShapeDtypeStruct((n, IN_DIM), _F32),
            jax.ShapeDtypeStruct((n, LATENT), _F32),
            jax.ShapeDtypeStruct((P_CORES, H1, H2), _F32),
            jax.ShapeDtypeStruct((P_CORES, H2, H1), _F32),
        ],
        compiler_params=pltpu.CompilerParams(
            dimension_semantics=("parallel", "arbitrary"),
            vmem_limit_bytes=56 * 1024 * 1024),
        name="sindy_forward",
        interpret=interpret,
    )(x, enc_w0, enc_b0, enc_w1, enc_b1, enc_w2, enc_b2,
      dec_w0, dec_b0, dec_w1, dec_b1, dec_w2, dec_b2, E_w, E_b,
      jnp.asarray(_SINDY_S), jnp.asarray(_SINDY_B))

    jet, jdt = pl.pallas_call(
        _jac_kernel,
        out_shape=[
            jax.ShapeDtypeStruct((IN_DIM, LATENT), _F32),
            jax.ShapeDtypeStruct((LATENT, IN_DIM), _F32),
        ],
        name="sindy_mean_jac",
        interpret=interpret,
    )(ge_parts, gd_parts, enc_w0, enc_w1, enc_w2, dec_w0, dec_w1, dec_w2)

    js = n // (P_CORES * BLK_STREAM)
    srow = lambda i, j: (i * js + j, 0)
    dz, dxb = pl.pallas_call(
        _stream_kernel,
        grid=(P_CORES, js),
        in_specs=[
            pl.BlockSpec((BLK_STREAM, IN_DIM), srow),
            pl.BlockSpec((BLK_STREAM, LATENT), srow),
            _full((IN_DIM, LATENT)),
            _full((LATENT, IN_DIM)),
        ],
        out_specs=[
            pl.BlockSpec((BLK_STREAM, LATENT), srow),
            pl.BlockSpec((BLK_STREAM, IN_DIM), srow),
        ],
        out_shape=[
            jax.ShapeDtypeStruct((n, LATENT), _F32),
            jax.ShapeDtypeStruct((n, IN_DIM), _F32),
        ],
        compiler_params=pltpu.CompilerParams(
            dimension_semantics=("parallel", "arbitrary"),
            vmem_limit_bytes=56 * 1024 * 1024),
        name="sindy_stream",
        interpret=interpret,
    )(dx, dzb, jet, jdt)

    return (z, dz, dzb, xb, dxb)


# speedup vs baseline: 2.1592x; 1.0028x over previous
import functools

import jax
import jax.numpy as jnp
import numpy as np
from jax.experimental import pallas as pl
from jax.experimental.pallas import tpu as pltpu

N_ROWS = 65536
IN_DIM = 512
H1, H2 = 256, 128
LATENT = 3
SINDY_DIM = 22

P_CORES = 1
BLK_FWD = 4096
BLK_STREAM = 4096

_F32 = jnp.float32
_BF16 = jnp.bfloat16


def _dot(a, b):
    return jnp.dot(a, b, preferred_element_type=_F32)


def _sindy_selectors():
    factors = [[] for _ in range(LATENT)]
    factors += [[i] for i in range(LATENT)]
    for i in range(LATENT):
        for j in range(i, LATENT):
            factors.append([i, j])
    for i in range(LATENT):
        for j in range(i, LATENT):
            for k in range(j, LATENT):
                factors.append([i, j, k])
    S = np.zeros((3, LATENT, SINDY_DIM), np.float32)
    b = np.zeros((3, SINDY_DIM), np.float32)
    for t, f in enumerate(factors):
        for m in range(3):
            if m < len(f):
                S[m, f[m], t] = 1.0
            else:
                b[m, t] = 1.0
    return S, b


def _fwd_kernel(x_ref, ew0, eb0, ew1, eb1, ew2, eb2,
                dw0, db0, dw1, db1, dw2, db2, Ew, Eb, S_ref, sb_ref,
                z_ref, xb_ref, dzb_ref, ge_ref, gd_ref):
    j = pl.program_id(1)

    x = x_ref[...]
    h0 = jax.nn.sigmoid(_dot(x, ew0[...]) + eb0[...])
    g0 = h0 * (1.0 - h0)
    h1 = jax.nn.sigmoid(_dot(h0, ew1[...]) + eb1[...])
    g1 = h1 * (1.0 - h1)
    z = _dot(h1, ew2[...]) + eb2[...]
    z_ref[...] = z

    ge_blk = jax.lax.dot_general(g0, g1, (((0,), (0,)), ((), ())),
                                 preferred_element_type=_F32)

    @pl.when(j == 0)
    def _():
        ge_ref[...] = ge_blk[None]

    @pl.when(j != 0)
    def _():
        ge_ref[...] += ge_blk[None]

    p0 = _dot(z, S_ref[0]) + sb_ref[0, :]
    p1 = _dot(z, S_ref[1]) + sb_ref[1, :]
    p2 = _dot(z, S_ref[2]) + sb_ref[2, :]
    theta = p0 * p1 * p2
    dzb_ref[...] = _dot(theta, Ew[...]) + Eb[...]

    hd0 = jax.nn.sigmoid(_dot(z, dw0[...]) + db0[...])
    gd0 = hd0 * (1.0 - hd0)
    hd1 = jax.nn.sigmoid(_dot(hd0, dw1[...]) + db1[...])
    gd1 = hd1 * (1.0 - hd1)
    xb_ref[...] = _dot(hd1, dw2[...]) + db2[...]

    gd_blk = jax.lax.dot_general(gd0, gd1, (((0,), (0,)), ((), ())),
                                 preferred_element_type=_F32)

    @pl.when(j == 0)
    def _():
        gd_ref[...] = gd_blk[None]

    @pl.when(j != 0)
    def _():
        gd_ref[...] += gd_blk[None]


def _jac_kernel(ge_ref, gd_ref, ew0, ew1, ew2, dw0, dw1, dw2,
                jet_ref, jdt_ref):
    inv_n = _F32(1.0 / N_ROWS)
    ge = jnp.sum(ge_ref[...], axis=0) * inv_n
    jet_ref[...] = _dot(ew0[...], _dot(ew1[...] * ge, ew2[...]))
    gd = jnp.sum(gd_ref[...], axis=0) * inv_n
    jdt_ref[...] = _dot(_dot(dw0[...], dw1[...] * gd), dw2[...])


def _stream_kernel(dx_ref, dzb_ref, jet_ref, jdt_ref, dz_ref, dxb_ref):
    dz_ref[...] = _dot(dx_ref[...], jet_ref[...])
    dxb_ref[...] = _dot(dzb_ref[...], jdt_ref[...])


def _full(shape):
    return pl.BlockSpec(shape, lambda *_: tuple(0 for _ in shape))


_SINDY_S, _SINDY_B = _sindy_selectors()


def kernel(x, dx, ddx, enc_w0, enc_b0, enc_w1, enc_b1, enc_w2, enc_b2,
           dec_w0, dec_b0, dec_w1, dec_b1, dec_w2, dec_b2, E_w, E_b,
           interpret=False):
    del ddx

    n = x.shape[0]
    jf = n // (P_CORES * BLK_FWD)
    row = lambda i, j: (i * jf + j, 0)

    z, xb, dzb, ge_parts, gd_parts = pl.pallas_call(
        _fwd_kernel,
        grid=(P_CORES, jf),
        in_specs=[
            pl.BlockSpec((BLK_FWD, IN_DIM), row),
            _full((IN_DIM, H1)), _full((H1,)),
            _full((H1, H2)), _full((H2,)),
            _full((H2, LATENT)), _full((LATENT,)),
            _full((LATENT, H2)), _full((H2,)),
            _full((H2, H1)), _full((H1,)),
            _full((H1, IN_DIM)), _full((IN_DIM,)),
            _full((SINDY_DIM, LATENT)), _full((LATENT,)),
            _full((3, LATENT, SINDY_DIM)), _full((3, SINDY_DIM)),
        ],
        out_specs=[
            pl.BlockSpec((BLK_FWD, LATENT), row),
            pl.BlockSpec((BLK_FWD, IN_DIM), row),
            pl.BlockSpec((BLK_FWD, LATENT), row),
            pl.BlockSpec((1, H1, H2), lambda i, j: (i, 0, 0)),
            pl.BlockSpec((1, H2, H1), lambda i, j: (i, 0, 0)),
        ],
        out_shape=[
            jax.ShapeDtypeStruct((n, LATENT), _F32),
            jax.ShapeDtypeStruct((n, IN_DIM), _F32),
            jax.ShapeDtypeStruct((n, LATENT), _F32),
            jax.ShapeDtypeStruct((P_CORES, H1, H2), _F32),
            jax.ShapeDtypeStruct((P_CORES, H2, H1), _F32),
        ],
        compiler_params=pltpu.CompilerParams(
            dimension_semantics=("parallel", "arbitrary"),
            vmem_limit_bytes=56 * 1024 * 1024),
        name="sindy_forward",
        interpret=interpret,
    )(x, enc_w0, enc_b0, enc_w1, enc_b1, enc_w2, enc_b2,
      dec_w0, dec_b0, dec_w1, dec_b1, dec_w2, dec_b2, E_w, E_b,
      jnp.asarray(_SINDY_S), jnp.asarray(_SINDY_B))

    jet, jdt = pl.pallas_call(
        _jac_kernel,
        out_shape=[
            jax.ShapeDtypeStruct((IN_DIM, LATENT), _F32),
            jax.ShapeDtypeStruct((LATENT, IN_DIM), _F32),
        ],
        name="sindy_mean_jac",
        interpret=interpret,
    )(ge_parts, gd_parts, enc_w0, enc_w1, enc_w2, dec_w0, dec_w1, dec_w2)

    js = n // (P_CORES * BLK_STREAM)
    srow = lambda i, j: (i * js + j, 0)
    dz, dxb = pl.pallas_call(
        _stream_kernel,
        grid=(P_CORES, js),
        in_specs=[
            pl.BlockSpec((BLK_STREAM, IN_DIM), srow),
            pl.BlockSpec((BLK_STREAM, LATENT), srow),
            _full((IN_DIM, LATENT)),
            _full((LATENT, IN_DIM)),
        ],
        out_specs=[
            pl.BlockSpec((BLK_STREAM, LATENT), srow),
            pl.BlockSpec((BLK_STREAM, IN_DIM), srow),
        ],
        out_shape=[
            jax.ShapeDtypeStruct((n, LATENT), _F32),
            jax.ShapeDtypeStruct((n, IN_DIM), _F32),
        ],
        compiler_params=pltpu.CompilerParams(
            dimension_semantics=("parallel", "arbitrary"),
            vmem_limit_bytes=56 * 1024 * 1024),
        name="sindy_stream",
        interpret=interpret,
    )(dx, dzb, jet, jdt)

    return (z, dz, dzb, xb, dxb)


# speedup vs baseline: 2.1689x; 1.0045x over previous
import functools

import jax
import jax.numpy as jnp
import numpy as np
from jax.experimental import pallas as pl
from jax.experimental.pallas import tpu as pltpu

N_ROWS = 65536
IN_DIM = 512
H1, H2 = 256, 128
LATENT = 3
SINDY_DIM = 22

P_CORES = 1
BLK_FWD = 4096
BLK_STREAM = 4096

_F32 = jnp.float32
_BF16 = jnp.bfloat16


def _dot(a, b):
    return jnp.dot(a, b, preferred_element_type=_F32)


def _sindy_selectors():
    factors = [[] for _ in range(LATENT)]
    factors += [[i] for i in range(LATENT)]
    for i in range(LATENT):
        for j in range(i, LATENT):
            factors.append([i, j])
    for i in range(LATENT):
        for j in range(i, LATENT):
            for k in range(j, LATENT):
                factors.append([i, j, k])
    S = np.zeros((3, LATENT, SINDY_DIM), np.float32)
    b = np.zeros((3, SINDY_DIM), np.float32)
    for t, f in enumerate(factors):
        for m in range(3):
            if m < len(f):
                S[m, f[m], t] = 1.0
            else:
                b[m, t] = 1.0
    return S, b


def _fwd_kernel(x_ref, ew0, eb0, ew1, eb1, ew2, eb2,
                dw0, db0, dw1, db1, dw2, db2, Ew, Eb, S_ref, sb_ref,
                z_ref, xb_ref, dzb_ref, ge_ref, gd_ref):
    j = pl.program_id(1)

    x = x_ref[...]
    h0 = jax.nn.sigmoid(_dot(x, ew0[...]) + eb0[...])
    g0 = h0 * (1.0 - h0)
    h1 = jax.nn.sigmoid(_dot(h0, ew1[...]) + eb1[...])
    g1 = h1 * (1.0 - h1)
    z = _dot(h1, ew2[...]) + eb2[...]
    z_ref[...] = z

    ge_blk = jax.lax.dot_general(g0, g1, (((0,), (0,)), ((), ())),
                                 preferred_element_type=_F32)

    @pl.when(j == 0)
    def _():
        ge_ref[...] = ge_blk[None]

    @pl.when(j != 0)
    def _():
        ge_ref[...] += ge_blk[None]

    p0 = _dot(z, S_ref[0]) + sb_ref[0, :]
    p1 = _dot(z, S_ref[1]) + sb_ref[1, :]
    p2 = _dot(z, S_ref[2]) + sb_ref[2, :]
    theta = p0 * p1 * p2
    dzb_ref[...] = _dot(theta, Ew[...]) + Eb[...]

    hd0 = jax.nn.sigmoid(_dot(z, dw0[...]) + db0[...])
    gd0 = hd0 * (1.0 - hd0)
    hd1 = jax.nn.sigmoid(_dot(hd0, dw1[...]) + db1[...])
    gd1 = hd1 * (1.0 - hd1)
    xb_ref[...] = _dot(hd1, dw2[...]) + db2[...]

    gd_blk = jax.lax.dot_general(gd0, gd1, (((0,), (0,)), ((), ())),
                                 preferred_element_type=_F32)

    @pl.when(j == 0)
    def _():
        gd_ref[...] = gd_blk[None]

    @pl.when(j != 0)
    def _():
        gd_ref[...] += gd_blk[None]


def _stream_kernel(dx_ref, dzb_ref, ge_ref, gd_ref,
                   ew0, ew1, ew2, dw0, dw1, dw2,
                   dz_ref, dxb_ref, jet_ref, jdt_ref):
    j = pl.program_id(1)

    @pl.when(j == 0)
    def _():
        inv_n = _F32(1.0 / N_ROWS)
        ge = jnp.sum(ge_ref[...], axis=0) * inv_n
        jet_ref[...] = _dot(ew0[...], _dot(ew1[...] * ge, ew2[...]))
        gd = jnp.sum(gd_ref[...], axis=0) * inv_n
        jdt_ref[...] = _dot(_dot(dw0[...], dw1[...] * gd), dw2[...])

    dz_ref[...] = _dot(dx_ref[...], jet_ref[...])
    dxb_ref[...] = _dot(dzb_ref[...], jdt_ref[...])


def _full(shape):
    return pl.BlockSpec(shape, lambda *_: tuple(0 for _ in shape))


_SINDY_S, _SINDY_B = _sindy_selectors()


def kernel(x, dx, ddx, enc_w0, enc_b0, enc_w1, enc_b1, enc_w2, enc_b2,
           dec_w0, dec_b0, dec_w1, dec_b1, dec_w2, dec_b2, E_w, E_b,
           interpret=False):
    del ddx

    n = x.shape[0]
    jf = n // (P_CORES * BLK_FWD)
    row = lambda i, j: (i * jf + j, 0)

    z, xb, dzb, ge_parts, gd_parts = pl.pallas_call(
        _fwd_kernel,
        grid=(P_CORES, jf),
        in_specs=[
            pl.BlockSpec((BLK_FWD, IN_DIM), row),
            _full((IN_DIM, H1)), _full((H1,)),
            _full((H1, H2)), _full((H2,)),
            _full((H2, LATENT)), _full((LATENT,)),
            _full((LATENT, H2)), _full((H2,)),
            _full((H2, H1)), _full((H1,)),
            _full((H1, IN_DIM)), _full((IN_DIM,)),
            _full((SINDY_DIM, LATENT)), _full((LATENT,)),
            _full((3, LATENT, SINDY_DIM)), _full((3, SINDY_DIM)),
        ],
        out_specs=[
            pl.BlockSpec((BLK_FWD, LATENT), row),
            pl.BlockSpec((BLK_FWD, IN_DIM), row),
            pl.BlockSpec((BLK_FWD, LATENT), row),
            pl.BlockSpec((1, H1, H2), lambda i, j: (i, 0, 0)),
            pl.BlockSpec((1, H2, H1), lambda i, j: (i, 0, 0)),
        ],
        out_shape=[
            jax.ShapeDtypeStruct((n, LATENT), _F32),
            jax.ShapeDtypeStruct((n, IN_DIM), _F32),
            jax.ShapeDtypeStruct((n, LATENT), _F32),
            jax.ShapeDtypeStruct((P_CORES, H1, H2), _F32),
            jax.ShapeDtypeStruct((P_CORES, H2, H1), _F32),
        ],
        compiler_params=pltpu.CompilerParams(
            dimension_semantics=("parallel", "arbitrary"),
            vmem_limit_bytes=56 * 1024 * 1024),
        name="sindy_forward",
        interpret=interpret,
    )(x, enc_w0, enc_b0, enc_w1, enc_b1, enc_w2, enc_b2,
      dec_w0, dec_b0, dec_w1, dec_b1, dec_w2, dec_b2, E_w, E_b,
      jnp.asarray(_SINDY_S), jnp.asarray(_SINDY_B))

    js = n // (P_CORES * BLK_STREAM)
    srow = lambda i, j: (i * js + j, 0)
    dz, dxb, _, _ = pl.pallas_call(
        _stream_kernel,
        grid=(P_CORES, js),
        in_specs=[
            pl.BlockSpec((BLK_STREAM, IN_DIM), srow),
            pl.BlockSpec((BLK_STREAM, LATENT), srow),
            _full((P_CORES, H1, H2)),
            _full((P_CORES, H2, H1)),
            _full((IN_DIM, H1)),
            _full((H1, H2)),
            _full((H2, LATENT)),
            _full((LATENT, H2)),
            _full((H2, H1)),
            _full((H1, IN_DIM)),
        ],
        out_specs=[
            pl.BlockSpec((BLK_STREAM, LATENT), srow),
            pl.BlockSpec((BLK_STREAM, IN_DIM), srow),
            _full((IN_DIM, LATENT)),
            _full((LATENT, IN_DIM)),
        ],
        out_shape=[
            jax.ShapeDtypeStruct((n, LATENT), _F32),
            jax.ShapeDtypeStruct((n, IN_DIM), _F32),
            jax.ShapeDtypeStruct((IN_DIM, LATENT), _F32),
            jax.ShapeDtypeStruct((LATENT, IN_DIM), _F32),
        ],
        compiler_params=pltpu.CompilerParams(
            dimension_semantics=("parallel", "arbitrary"),
            vmem_limit_bytes=56 * 1024 * 1024),
        name="sindy_stream",
        interpret=interpret,
    )(dx, dzb, ge_parts, gd_parts,
      enc_w0, enc_w1, enc_w2, dec_w0, dec_w1, dec_w2)

    return (z, dz, dzb, xb, dxb)


# speedup vs baseline: 2.1907x; 1.0101x over previous
import functools

import jax
import jax.numpy as jnp
import numpy as np
from jax.experimental import pallas as pl
from jax.experimental.pallas import tpu as pltpu

N_ROWS = 65536
IN_DIM = 512
H1, H2 = 256, 128
LATENT = 3
SINDY_DIM = 22

P_CORES = 1
BLK_FWD = 4096
BLK_STREAM = 4096

_F32 = jnp.float32
_BF16 = jnp.bfloat16


def _dot(a, b):
    return jnp.dot(a, b, preferred_element_type=_F32)


def _sindy_selectors():
    factors = [[] for _ in range(LATENT)]
    factors += [[i] for i in range(LATENT)]
    for i in range(LATENT):
        for j in range(i, LATENT):
            factors.append([i, j])
    for i in range(LATENT):
        for j in range(i, LATENT):
            for k in range(j, LATENT):
                factors.append([i, j, k])
    S = np.zeros((3, LATENT, SINDY_DIM), np.float32)
    b = np.zeros((3, SINDY_DIM), np.float32)
    for t, f in enumerate(factors):
        for m in range(3):
            if m < len(f):
                S[m, f[m], t] = 1.0
            else:
                b[m, t] = 1.0
    return S, b


def _fwd_kernel(x_ref, ew0, eb0, ew1, eb1, ew2, eb2,
                dw0, db0, dw1, db1, dw2, db2, Ew, Eb, S_ref, sb_ref,
                z_ref, xb_ref, dzb_ref, ge_ref, gd_ref):
    j = pl.program_id(1)

    x = x_ref[...].astype(_BF16)
    h0 = jax.nn.sigmoid(_dot(x, ew0[...].astype(_BF16)) + eb0[...])
    h0b = h0.astype(_BF16)
    g0 = h0b * (1.0 - h0b)
    h1 = jax.nn.sigmoid(_dot(h0b, ew1[...].astype(_BF16)) + eb1[...])
    h1b = h1.astype(_BF16)
    g1 = h1b * (1.0 - h1b)
    z = _dot(h1b, ew2[...].astype(_BF16)) + eb2[...]
    z_ref[...] = z

    ge_blk = jax.lax.dot_general(g0, g1, (((0,), (0,)), ((), ())),
                                 preferred_element_type=_F32)

    @pl.when(j == 0)
    def _():
        ge_ref[...] = ge_blk[None]

    @pl.when(j != 0)
    def _():
        ge_ref[...] += ge_blk[None]

    p0 = _dot(z, S_ref[0]) + sb_ref[0, :]
    p1 = _dot(z, S_ref[1]) + sb_ref[1, :]
    p2 = _dot(z, S_ref[2]) + sb_ref[2, :]
    theta = p0 * p1 * p2
    dzb_ref[...] = _dot(theta, Ew[...]) + Eb[...]

    hd0 = jax.nn.sigmoid(_dot(z.astype(_BF16), dw0[...].astype(_BF16))
                         + db0[...])
    hd0b = hd0.astype(_BF16)
    gd0 = hd0b * (1.0 - hd0b)
    hd1 = jax.nn.sigmoid(_dot(hd0b, dw1[...].astype(_BF16)) + db1[...])
    hd1b = hd1.astype(_BF16)
    gd1 = hd1b * (1.0 - hd1b)
    xb_ref[...] = _dot(hd1b, dw2[...].astype(_BF16)) + db2[...]

    gd_blk = jax.lax.dot_general(gd0, gd1, (((0,), (0,)), ((), ())),
                                 preferred_element_type=_F32)

    @pl.when(j == 0)
    def _():
        gd_ref[...] = gd_blk[None]

    @pl.when(j != 0)
    def _():
        gd_ref[...] += gd_blk[None]


def _stream_kernel(dx_ref, dzb_ref, ge_ref, gd_ref,
                   ew0, ew1, ew2, dw0, dw1, dw2,
                   dz_ref, dxb_ref, jet_ref, jdt_ref):
    j = pl.program_id(1)

    @pl.when(j == 0)
    def _():
        inv_n = _F32(1.0 / N_ROWS)
        ge = jnp.sum(ge_ref[...], axis=0) * inv_n
        jet_ref[...] = _dot(ew0[...], _dot(ew1[...] * ge, ew2[...]))
        gd = jnp.sum(gd_ref[...], axis=0) * inv_n
        jdt_ref[...] = _dot(_dot(dw0[...], dw1[...] * gd), dw2[...])

    dz_ref[...] = _dot(dx_ref[...], jet_ref[...])
    dxb_ref[...] = _dot(dzb_ref[...], jdt_ref[...])


def _full(shape):
    return pl.BlockSpec(shape, lambda *_: tuple(0 for _ in shape))


_SINDY_S, _SINDY_B = _sindy_selectors()


def kernel(x, dx, ddx, enc_w0, enc_b0, enc_w1, enc_b1, enc_w2, enc_b2,
           dec_w0, dec_b0, dec_w1, dec_b1, dec_w2, dec_b2, E_w, E_b,
           interpret=False):
    del ddx

    n = x.shape[0]
    jf = n // (P_CORES * BLK_FWD)
    row = lambda i, j: (i * jf + j, 0)

    z, xb, dzb, ge_parts, gd_parts = pl.pallas_call(
        _fwd_kernel,
        grid=(P_CORES, jf),
        in_specs=[
            pl.BlockSpec((BLK_FWD, IN_DIM), row),
            _full((IN_DIM, H1)), _full((H1,)),
            _full((H1, H2)), _full((H2,)),
            _full((H2, LATENT)), _full((LATENT,)),
            _full((LATENT, H2)), _full((H2,)),
            _full((H2, H1)), _full((H1,)),
            _full((H1, IN_DIM)), _full((IN_DIM,)),
            _full((SINDY_DIM, LATENT)), _full((LATENT,)),
            _full((3, LATENT, SINDY_DIM)), _full((3, SINDY_DIM)),
        ],
        out_specs=[
            pl.BlockSpec((BLK_FWD, LATENT), row),
            pl.BlockSpec((BLK_FWD, IN_DIM), row),
            pl.BlockSpec((BLK_FWD, LATENT), row),
            pl.BlockSpec((1, H1, H2), lambda i, j: (i, 0, 0)),
            pl.BlockSpec((1, H2, H1), lambda i, j: (i, 0, 0)),
        ],
        out_shape=[
            jax.ShapeDtypeStruct((n, LATENT), _F32),
            jax.ShapeDtypeStruct((n, IN_DIM), _F32),
            jax.ShapeDtypeStruct((n, LATENT), _F32),
            jax.ShapeDtypeStruct((P_CORES, H1, H2), _F32),
            jax.ShapeDtypeStruct((P_CORES, H2, H1), _F32),
        ],
        compiler_params=pltpu.CompilerParams(
            dimension_semantics=("parallel", "arbitrary"),
            vmem_limit_bytes=56 * 1024 * 1024),
        name="sindy_forward",
        interpret=interpret,
    )(x, enc_w0, enc_b0, enc_w1, enc_b1, enc_w2, enc_b2,
      dec_w0, dec_b0, dec_w1, dec_b1, dec_w2, dec_b2, E_w, E_b,
      jnp.asarray(_SINDY_S), jnp.asarray(_SINDY_B))

    js = n // (P_CORES * BLK_STREAM)
    srow = lambda i, j: (i * js + j, 0)
    dz, dxb, _, _ = pl.pallas_call(
        _stream_kernel,
        grid=(P_CORES, js),
        in_specs=[
            pl.BlockSpec((BLK_STREAM, IN_DIM), srow),
            pl.BlockSpec((BLK_STREAM, LATENT), srow),
            _full((P_CORES, H1, H2)),
            _full((P_CORES, H2, H1)),
            _full((IN_DIM, H1)),
            _full((H1, H2)),
            _full((H2, LATENT)),
            _full((LATENT, H2)),
            _full((H2, H1)),
            _full((H1, IN_DIM)),
        ],
        out_specs=[
            pl.BlockSpec((BLK_STREAM, LATENT), srow),
            pl.BlockSpec((BLK_STREAM, IN_DIM), srow),
            _full((IN_DIM, LATENT)),
            _full((LATENT, IN_DIM)),
        ],
        out_shape=[
            jax.ShapeDtypeStruct((n, LATENT), _F32),
            jax.ShapeDtypeStruct((n, IN_DIM), _F32),
            jax.ShapeDtypeStruct((IN_DIM, LATENT), _F32),
            jax.ShapeDtypeStruct((LATENT, IN_DIM), _F32),
        ],
        compiler_params=pltpu.CompilerParams(
            dimension_semantics=("parallel", "arbitrary"),
            vmem_limit_bytes=56 * 1024 * 1024),
        name="sindy_stream",
        interpret=interpret,
    )(dx, dzb, ge_parts, gd_parts,
      enc_w0, enc_w1, enc_w2, dec_w0, dec_w1, dec_w2)

    return (z, dz, dzb, xb, dxb)


# speedup vs baseline: 2.2028x; 1.0055x over previous
import functools

import jax
import jax.numpy as jnp
import numpy as np
from jax.experimental import pallas as pl
from jax.experimental.pallas import tpu as pltpu

N_ROWS = 65536
IN_DIM = 512
H1, H2 = 256, 128
LATENT = 3
SINDY_DIM = 22

P_CORES = 1
BLK_FWD = 4096
BLK_STREAM = 4096

_F32 = jnp.float32
_BF16 = jnp.bfloat16


def _dot(a, b):
    return jnp.dot(a, b, preferred_element_type=_F32)


def _sig(pre):
    t = jnp.tanh(pre * 0.5)
    return 0.5 * t + 0.5, 0.25 - 0.25 * (t * t)


def _sindy_selectors():
    factors = [[] for _ in range(LATENT)]
    factors += [[i] for i in range(LATENT)]
    for i in range(LATENT):
        for j in range(i, LATENT):
            factors.append([i, j])
    for i in range(LATENT):
        for j in range(i, LATENT):
            for k in range(j, LATENT):
                factors.append([i, j, k])
    S = np.zeros((3, LATENT, SINDY_DIM), np.float32)
    b = np.zeros((3, SINDY_DIM), np.float32)
    for t, f in enumerate(factors):
        for m in range(3):
            if m < len(f):
                S[m, f[m], t] = 1.0
            else:
                b[m, t] = 1.0
    return S, b


def _fwd_kernel(x_ref, ew0, eb0, ew1, eb1, ew2, eb2,
                dw0, db0, dw1, db1, dw2, db2, Ew, Eb, S_ref, sb_ref,
                z_ref, xb_ref, dzb_ref, ge_ref, gd_ref):
    j = pl.program_id(1)

    x = x_ref[...].astype(_BF16)
    h0, g0f = _sig(_dot(x, ew0[...].astype(_BF16)) + eb0[...])
    h0b = h0.astype(_BF16)
    g0 = g0f.astype(_BF16)
    h1, g1f = _sig(_dot(h0b, ew1[...].astype(_BF16)) + eb1[...])
    h1b = h1.astype(_BF16)
    g1 = g1f.astype(_BF16)
    z = _dot(h1b, ew2[...].astype(_BF16)) + eb2[...]
    z_ref[...] = z

    ge_blk = jax.lax.dot_general(g0, g1, (((0,), (0,)), ((), ())),
                                 preferred_element_type=_F32)

    @pl.when(j == 0)
    def _():
        ge_ref[...] = ge_blk[None]

    @pl.when(j != 0)
    def _():
        ge_ref[...] += ge_blk[None]

    p0 = _dot(z, S_ref[0]) + sb_ref[0, :]
    p1 = _dot(z, S_ref[1]) + sb_ref[1, :]
    p2 = _dot(z, S_ref[2]) + sb_ref[2, :]
    theta = p0 * p1 * p2
    dzb_ref[...] = _dot(theta, Ew[...]) + Eb[...]

    hd0, gd0f = _sig(_dot(z.astype(_BF16), dw0[...].astype(_BF16))
                     + db0[...])
    hd0b = hd0.astype(_BF16)
    gd0 = gd0f.astype(_BF16)
    hd1, gd1f = _sig(_dot(hd0b, dw1[...].astype(_BF16)) + db1[...])
    hd1b = hd1.astype(_BF16)
    gd1 = gd1f.astype(_BF16)
    xb_ref[...] = _dot(hd1b, dw2[...].astype(_BF16)) + db2[...]

    gd_blk = jax.lax.dot_general(gd0, gd1, (((0,), (0,)), ((), ())),
                                 preferred_element_type=_F32)

    @pl.when(j == 0)
    def _():
        gd_ref[...] = gd_blk[None]

    @pl.when(j != 0)
    def _():
        gd_ref[...] += gd_blk[None]


def _stream_kernel(dx_ref, dzb_ref, ge_ref, gd_ref,
                   ew0, ew1, ew2, dw0, dw1, dw2,
                   dz_ref, dxb_ref, jet_ref, jdt_ref):
    j = pl.program_id(1)

    @pl.when(j == 0)
    def _():
        inv_n = _F32(1.0 / N_ROWS)
        ge = jnp.sum(ge_ref[...], axis=0) * inv_n
        jet_ref[...] = _dot(ew0[...], _dot(ew1[...] * ge, ew2[...]))
        gd = jnp.sum(gd_ref[...], axis=0) * inv_n
        jdt_ref[...] = _dot(_dot(dw0[...], dw1[...] * gd), dw2[...])

    dz_ref[...] = _dot(dx_ref[...], jet_ref[...])
    dxb_ref[...] = _dot(dzb_ref[...], jdt_ref[...])


def _full(shape):
    return pl.BlockSpec(shape, lambda *_: tuple(0 for _ in shape))


_SINDY_S, _SINDY_B = _sindy_selectors()


def kernel(x, dx, ddx, enc_w0, enc_b0, enc_w1, enc_b1, enc_w2, enc_b2,
           dec_w0, dec_b0, dec_w1, dec_b1, dec_w2, dec_b2, E_w, E_b,
           interpret=False):
    del ddx

    n = x.shape[0]
    jf = n // (P_CORES * BLK_FWD)
    row = lambda i, j: (i * jf + j, 0)

    z, xb, dzb, ge_parts, gd_parts = pl.pallas_call(
        _fwd_kernel,
        grid=(P_CORES, jf),
        in_specs=[
            pl.BlockSpec((BLK_FWD, IN_DIM), row),
            _full((IN_DIM, H1)), _full((H1,)),
            _full((H1, H2)), _full((H2,)),
            _full((H2, LATENT)), _full((LATENT,)),
            _full((LATENT, H2)), _full((H2,)),
            _full((H2, H1)), _full((H1,)),
            _full((H1, IN_DIM)), _full((IN_DIM,)),
            _full((SINDY_DIM, LATENT)), _full((LATENT,)),
            _full((3, LATENT, SINDY_DIM)), _full((3, SINDY_DIM)),
        ],
        out_specs=[
            pl.BlockSpec((BLK_FWD, LATENT), row),
            pl.BlockSpec((BLK_FWD, IN_DIM), row),
            pl.BlockSpec((BLK_FWD, LATENT), row),
            pl.BlockSpec((1, H1, H2), lambda i, j: (i, 0, 0)),
            pl.BlockSpec((1, H2, H1), lambda i, j: (i, 0, 0)),
        ],
        out_shape=[
            jax.ShapeDtypeStruct((n, LATENT), _F32),
            jax.ShapeDtypeStruct((n, IN_DIM), _F32),
            jax.ShapeDtypeStruct((n, LATENT), _F32),
            jax.ShapeDtypeStruct((P_CORES, H1, H2), _F32),
            jax.ShapeDtypeStruct((P_CORES, H2, H1), _F32),
        ],
        compiler_params=pltpu.CompilerParams(
            dimension_semantics=("parallel", "arbitrary"),
            vmem_limit_bytes=56 * 1024 * 1024),
        name="sindy_forward",
        interpret=interpret,
    )(x, enc_w0, enc_b0, enc_w1, enc_b1, enc_w2, enc_b2,
      dec_w0, dec_b0, dec_w1, dec_b1, dec_w2, dec_b2, E_w, E_b,
      jnp.asarray(_SINDY_S), jnp.asarray(_SINDY_B))

    js = n // (P_CORES * BLK_STREAM)
    srow = lambda i, j: (i * js + j, 0)
    dz, dxb, _, _ = pl.pallas_call(
        _stream_kernel,
        grid=(P_CORES, js),
        in_specs=[
            pl.BlockSpec((BLK_STREAM, IN_DIM), srow),
            pl.BlockSpec((BLK_STREAM, LATENT), srow),
            _full((P_CORES, H1, H2)),
            _full((P_CORES, H2, H1)),
            _full((IN_DIM, H1)),
            _full((H1, H2)),
            _full((H2, LATENT)),
            _full((LATENT, H2)),
            _full((H2, H1)),
            _full((H1, IN_DIM)),
        ],
        out_specs=[
            pl.BlockSpec((BLK_STREAM, LATENT), srow),
            pl.BlockSpec((BLK_STREAM, IN_DIM), srow),
            _full((IN_DIM, LATENT)),
            _full((LATENT, IN_DIM)),
        ],
        out_shape=[
            jax.ShapeDtypeStruct((n, LATENT), _F32),
            jax.ShapeDtypeStruct((n, IN_DIM), _F32),
            jax.ShapeDtypeStruct((IN_DIM, LATENT), _F32),
            jax.ShapeDtypeStruct((LATENT, IN_DIM), _F32),
        ],
        compiler_params=pltpu.CompilerParams(
            dimension_semantics=("parallel", "arbitrary"),
            vmem_limit_bytes=56 * 1024 * 1024),
        name="sindy_stream",
        interpret=interpret,
    )(dx, dzb, ge_parts, gd_parts,
      enc_w0, enc_w1, enc_w2, dec_w0, dec_w1, dec_w2)

    return (z, dz, dzb, xb, dxb)


# speedup vs baseline: 3.0358x; 1.3782x over previous
import functools

import jax
import jax.numpy as jnp
import numpy as np
from jax.experimental import pallas as pl
from jax.experimental.pallas import tpu as pltpu

N_ROWS = 65536
IN_DIM = 512
H1, H2 = 256, 128
LATENT = 3
SINDY_DIM = 22

P_CORES = 1
BLK_FWD = 4096
BLK_STREAM = 4096

_F32 = jnp.float32
_BF16 = jnp.bfloat16


def _dot(a, b):
    return jnp.dot(a, b, preferred_element_type=_F32)


def _sig(pre):
    t = jnp.tanh(pre * 0.5)
    return 0.5 * t + 0.5, 0.25 - 0.25 * (t * t)


def _sindy_selectors():
    factors = [[] for _ in range(LATENT)]
    factors += [[i] for i in range(LATENT)]
    for i in range(LATENT):
        for j in range(i, LATENT):
            factors.append([i, j])
    for i in range(LATENT):
        for j in range(i, LATENT):
            for k in range(j, LATENT):
                factors.append([i, j, k])
    S = np.zeros((3, LATENT, SINDY_DIM), np.float32)
    b = np.zeros((3, SINDY_DIM), np.float32)
    for t, f in enumerate(factors):
        for m in range(3):
            if m < len(f):
                S[m, f[m], t] = 1.0
            else:
                b[m, t] = 1.0
    return S, b


def _fwd_kernel(x_ref, ew0, eb0, ew1, eb1, ew2, eb2_col,
                dw0, db0, dw1, db1, dw2, db2, Ew, Eb_col, S_ref, sb_col,
                z_ref, xb_ref, dzb_ref, ge_ref, gd_ref):
    j = pl.program_id(1)

    x = x_ref[...].astype(_BF16)
    h0, g0f = _sig(_dot(x, ew0[...].astype(_BF16)) + eb0[...])
    h0b = h0.astype(_BF16)
    g0 = g0f.astype(_BF16)
    h1, g1f = _sig(_dot(h0b, ew1[...].astype(_BF16)) + eb1[...])
    h1b = h1.astype(_BF16)
    g1 = g1f.astype(_BF16)
    zt = jax.lax.dot_general(ew2[...].astype(_BF16), h1b,
                             (((0,), (1,)), ((), ())),
                             preferred_element_type=_F32) + eb2_col[...]
    z_ref[...] = zt

    ge_blk = jax.lax.dot_general(g0, g1, (((0,), (0,)), ((), ())),
                                 preferred_element_type=_F32)

    @pl.when(j == 0)
    def _():
        ge_ref[...] = ge_blk[None]

    @pl.when(j != 0)
    def _():
        ge_ref[...] += ge_blk[None]

    p0 = jax.lax.dot_general(S_ref[0], zt, (((0,), (0,)), ((), ())),
                             preferred_element_type=_F32) + sb_col[0]
    p1 = jax.lax.dot_general(S_ref[1], zt, (((0,), (0,)), ((), ())),
                             preferred_element_type=_F32) + sb_col[1]
    p2 = jax.lax.dot_general(S_ref[2], zt, (((0,), (0,)), ((), ())),
                             preferred_element_type=_F32) + sb_col[2]
    theta_t = p0 * p1 * p2
    dzb_ref[...] = jax.lax.dot_general(
        Ew[...], theta_t, (((0,), (0,)), ((), ())),
        preferred_element_type=_F32) + Eb_col[...]

    hd0, gd0f = _sig(jax.lax.dot_general(
        zt.astype(_BF16), dw0[...].astype(_BF16),
        (((0,), (0,)), ((), ())), preferred_element_type=_F32)
        + db0[...])
    hd0b = hd0.astype(_BF16)
    gd0 = gd0f.astype(_BF16)
    hd1, gd1f = _sig(_dot(hd0b, dw1[...].astype(_BF16)) + db1[...])
    hd1b = hd1.astype(_BF16)
    gd1 = gd1f.astype(_BF16)
    xb_ref[...] = _dot(hd1b, dw2[...].astype(_BF16)) + db2[...]

    gd_blk = jax.lax.dot_general(gd0, gd1, (((0,), (0,)), ((), ())),
                                 preferred_element_type=_F32)

    @pl.when(j == 0)
    def _():
        gd_ref[...] = gd_blk[None]

    @pl.when(j != 0)
    def _():
        gd_ref[...] += gd_blk[None]


def _stream_kernel(dx_ref, dzb_ref, ge_ref, gd_ref,
                   ew0, ew1, ew2, dw0, dw1, dw2,
                   dz_ref, dxb_ref, jet_ref, jdt_ref):
    j = pl.program_id(1)

    @pl.when(j == 0)
    def _():
        inv_n = _F32(1.0 / N_ROWS)
        ge = jnp.sum(ge_ref[...], axis=0) * inv_n
        jet_ref[...] = _dot(ew0[...], _dot(ew1[...] * ge, ew2[...]))
        gd = jnp.sum(gd_ref[...], axis=0) * inv_n
        jdt_ref[...] = _dot(_dot(dw0[...], dw1[...] * gd), dw2[...])

    dz_ref[...] = jax.lax.dot_general(
        jet_ref[...], dx_ref[...], (((0,), (1,)), ((), ())),
        preferred_element_type=_F32)
    dxb_ref[...] = jax.lax.dot_general(
        dzb_ref[...], jdt_ref[...], (((0,), (0,)), ((), ())),
        preferred_element_type=_F32)


def _full(shape):
    return pl.BlockSpec(shape, lambda *_: tuple(0 for _ in shape))


_SINDY_S, _SINDY_B = _sindy_selectors()


def kernel(x, dx, ddx, enc_w0, enc_b0, enc_w1, enc_b1, enc_w2, enc_b2,
           dec_w0, dec_b0, dec_w1, dec_b1, dec_w2, dec_b2, E_w, E_b,
           interpret=False):
    del ddx

    n = x.shape[0]
    jf = n // (P_CORES * BLK_FWD)
    row = lambda i, j: (i * jf + j, 0)
    col = lambda i, j: (0, i * jf + j)
    eb2_col = enc_b2[:, None]
    eb_col = E_b[:, None]
    sb_col = _SINDY_B[:, :, None]

    z, xb, dzb, ge_parts, gd_parts = pl.pallas_call(
        _fwd_kernel,
        grid=(P_CORES, jf),
        in_specs=[
            pl.BlockSpec((BLK_FWD, IN_DIM), row),
            _full((IN_DIM, H1)), _full((H1,)),
            _full((H1, H2)), _full((H2,)),
            _full((H2, LATENT)), _full((LATENT, 1)),
            _full((LATENT, H2)), _full((H2,)),
            _full((H2, H1)), _full((H1,)),
            _full((H1, IN_DIM)), _full((IN_DIM,)),
            _full((SINDY_DIM, LATENT)), _full((LATENT, 1)),
            _full((3, LATENT, SINDY_DIM)), _full((3, SINDY_DIM, 1)),
        ],
        out_specs=[
            pl.BlockSpec((LATENT, BLK_FWD), col),
            pl.BlockSpec((BLK_FWD, IN_DIM), row),
            pl.BlockSpec((LATENT, BLK_FWD), col),
            pl.BlockSpec((1, H1, H2), lambda i, j: (i, 0, 0)),
            pl.BlockSpec((1, H2, H1), lambda i, j: (i, 0, 0)),
        ],
        out_shape=[
            jax.ShapeDtypeStruct((LATENT, n), _F32),
            jax.ShapeDtypeStruct((n, IN_DIM), _F32),
            jax.ShapeDtypeStruct((LATENT, n), _F32),
            jax.ShapeDtypeStruct((P_CORES, H1, H2), _F32),
            jax.ShapeDtypeStruct((P_CORES, H2, H1), _F32),
        ],
        compiler_params=pltpu.CompilerParams(
            dimension_semantics=("parallel", "arbitrary"),
            vmem_limit_bytes=56 * 1024 * 1024),
        name="sindy_forward",
        interpret=interpret,
    )(x, enc_w0, enc_b0, enc_w1, enc_b1, enc_w2, eb2_col,
      dec_w0, dec_b0, dec_w1, dec_b1, dec_w2, dec_b2, E_w, eb_col,
      jnp.asarray(_SINDY_S), jnp.asarray(sb_col))

    js = n // (P_CORES * BLK_STREAM)
    srow = lambda i, j: (i * js + j, 0)
    scol = lambda i, j: (0, i * js + j)
    dz, dxb, _, _ = pl.pallas_call(
        _stream_kernel,
        grid=(P_CORES, js),
        in_specs=[
            pl.BlockSpec((BLK_STREAM, IN_DIM), srow),
            pl.BlockSpec((LATENT, BLK_STREAM), scol),
            _full((P_CORES, H1, H2)),
            _full((P_CORES, H2, H1)),
            _full((IN_DIM, H1)),
            _full((H1, H2)),
            _full((H2, LATENT)),
            _full((LATENT, H2)),
            _full((H2, H1)),
            _full((H1, IN_DIM)),
        ],
        out_specs=[
            pl.BlockSpec((LATENT, BLK_STREAM), scol),
            pl.BlockSpec((BLK_STREAM, IN_DIM), srow),
            _full((IN_DIM, LATENT)),
            _full((LATENT, IN_DIM)),
        ],
        out_shape=[
            jax.ShapeDtypeStruct((LATENT, n), _F32),
            jax.ShapeDtypeStruct((n, IN_DIM), _F32),
            jax.ShapeDtypeStruct((IN_DIM, LATENT), _F32),
            jax.ShapeDtypeStruct((LATENT, IN_DIM), _F32),
        ],
        compiler_params=pltpu.CompilerParams(
            dimension_semantics=("parallel", "arbitrary"),
            vmem_limit_bytes=56 * 1024 * 1024),
        name="sindy_stream",
        interpret=interpret,
    )(dx, dzb, ge_parts, gd_parts,
      enc_w0, enc_w1, enc_w2, dec_w0, dec_w1, dec_w2)

    return (z.T, dz.T, dzb.T, xb, dxb)
